# baseline (speedup 1.0000x reference)
import functools

import jax
import jax.numpy as jnp
from jax.experimental import pallas as pl
from jax.experimental.pallas import tpu as pltpu

K = 16
DILATION = 2
KSEL = K * DILATION


def _dist_body(xl_ref, xr_ref, d_ref, g_ref):
    a = xl_ref[0]
    b = xr_ref[0]
    mm = jax.lax.dot_general(
        a, b, (((1,), (1,)), ((), ())), preferred_element_type=jnp.float32
    )
    asq = jnp.sum(a * a, axis=1, keepdims=True)
    bsq = jnp.sum(b * b, axis=1, keepdims=True)
    d = (asq + (-2.0 * mm)) + bsq.T
    d_ref[0] = d
    bi, bj = d.shape
    g_ref[0] = jnp.min(d.reshape(bi, bj // 128, 128), axis=2)


@functools.partial(jax.jit, static_argnames=("bi",))
def _pairwise_dist(xt, bi=256):
    B, N, C = xt.shape
    grid = (B, N // bi)
    return pl.pallas_call(
        _dist_body,
        grid=grid,
        in_specs=[
            pl.BlockSpec((1, bi, C), lambda b, i: (b, i, 0)),
            pl.BlockSpec((1, N, C), lambda b, i: (b, 0, 0)),
        ],
        out_specs=[
            pl.BlockSpec((1, bi, N), lambda b, i: (b, i, 0)),
            pl.BlockSpec((1, bi, N // 128), lambda b, i: (b, i, 0)),
        ],
        out_shape=[
            jax.ShapeDtypeStruct((B, N, N), jnp.float32),
            jax.ShapeDtypeStruct((B, N, N // 128), jnp.float32),
        ],
        compiler_params=pltpu.CompilerParams(
            dimension_semantics=("parallel", "arbitrary"),
        ),
    )(xt, xt)


def kernel(x):
    B, C, N, _ = x.shape
    xt = jnp.squeeze(jnp.swapaxes(x, 2, 1), -1)
    dist, _gmins = _pairwise_dist(xt)
    _, nn_idx = jax.lax.top_k(-dist, KSEL)
    nn_idx = nn_idx[:, :, ::DILATION]
    center = jnp.broadcast_to(
        jnp.arange(N, dtype=nn_idx.dtype)[None, :, None], (B, N, K)
    )
    return jnp.stack((nn_idx, center), axis=0)


# speedup vs baseline: 7.6304x; 7.6304x over previous
import functools

import jax
import jax.numpy as jnp
from jax import lax
from jax.experimental import pallas as pl
from jax.experimental.pallas import tpu as pltpu
from jax.experimental.pallas import tpu_sc as plsc

K = 16
DILATION = 2
KSEL = K * DILATION

NC = 2
NS = 16
NW = NC * NS
INT_MAX = 2147483647


def _dist_body(xl_ref, xr_ref, d_ref, g_ref):
    a = xl_ref[0]
    b = xr_ref[0]
    mm = jax.lax.dot_general(
        a, b, (((1,), (1,)), ((), ())), preferred_element_type=jnp.float32
    )
    asq = jnp.sum(a * a, axis=1, keepdims=True)
    bsq = jnp.sum(b * b, axis=1, keepdims=True)
    d = (asq + (-2.0 * mm)) + bsq.T
    d_ref[0] = d
    bi, n = d.shape
    g_ref[0] = jnp.min(d.reshape(bi, 32, n // 32), axis=2)


@functools.partial(jax.jit, static_argnames=("bi",))
def _pairwise_dist(xt, bi=256):
    B, N, C = xt.shape
    grid = (B, N // bi)
    return pl.pallas_call(
        _dist_body,
        grid=grid,
        in_specs=[
            pl.BlockSpec((1, bi, C), lambda b, i: (b, i, 0)),
            pl.BlockSpec((1, N, C), lambda b, i: (b, 0, 0)),
        ],
        out_specs=[
            pl.BlockSpec((1, bi, N), lambda b, i: (b, i, 0)),
            pl.BlockSpec((1, bi, 32), lambda b, i: (b, i, 0)),
        ],
        out_shape=[
            jax.ShapeDtypeStruct((B, N, N), jnp.float32),
            jax.ShapeDtypeStruct((B, N, 32), jnp.float32),
        ],
        compiler_params=pltpu.CompilerParams(
            dimension_semantics=("parallel", "arbitrary"),
        ),
    )(xt, xt)


def _make_select(num_rows, n):
    rpw = num_rows // NW
    nvreg = n // 16
    cap = 1024 + 32
    mesh = plsc.VectorSubcoreMesh(
        core_axis_name="c", subcore_axis_name="s", num_cores=NC, num_subcores=NS
    )

    @functools.partial(
        pl.kernel,
        out_type=jax.ShapeDtypeStruct((num_rows * K,), jnp.int32),
        mesh=mesh,
        compiler_params=pltpu.CompilerParams(needs_layout_passes=False),
        scratch_types=[
            pltpu.VMEM((n,), jnp.float32),
            pltpu.VMEM((n,), jnp.float32),
            pltpu.VMEM((rpw * 32,), jnp.float32),
            pltpu.VMEM((cap,), jnp.int32),
            pltpu.VMEM((32,), jnp.int32),
            pltpu.VMEM((rpw * K,), jnp.int32),
            pltpu.SemaphoreType.DMA,
            pltpu.SemaphoreType.DMA,
        ],
    )
    def select(dist_hbm, gmins_hbm, out_hbm, buf_a, buf_b, gall, cand, obuf,
               oall, sem_a, sem_b):
        wid = lax.axis_index("s") * NC + lax.axis_index("c")
        base = wid * rpw
        iota16 = lax.broadcasted_iota(jnp.int32, (16,), 0)
        inf_v = jnp.full((16,), jnp.inf, jnp.float32)
        imax_v = jnp.full((16,), INT_MAX, jnp.int32)

        pltpu.sync_copy(gmins_hbm.at[pl.ds(base * 32, rpw * 32)], gall)
        pltpu.async_copy(dist_hbm.at[pl.ds(base * n, n)], buf_a, sem_a)

        def process(buf, r):
            gm = jnp.maximum(
                gall[pl.ds(r * 32, 16)], gall[pl.ds(r * 32 + 16, 16)]
            )
            theta = jnp.max(gm, axis=0)
            theta_v = jnp.full((16,), theta, jnp.float32)

            def scan_body(g, cnt):
                x = buf[pl.ds(g * 16, 16)]
                m = x <= theta_v
                plsc.store_compressed(
                    cand.at[pl.ds(cnt, 16)], iota16 + g * 16, mask=m
                )
                return cnt + jnp.sum(m.astype(jnp.int32), axis=0)

            cnt = lax.fori_loop(0, nvreg, scan_body, jnp.int32(0), unroll=4)

            nb = (cnt + 15) >> 4
            cnt_v = jnp.full((16,), cnt, jnp.int32)

            def sel_body(t, carry):
                c0v, c0i, c1v, c1i = carry
                off = t * 16
                idx = cand[pl.ds(off, 16)]
                valid = (iota16 + off) < cnt_v
                idx_safe = jnp.minimum(jnp.maximum(idx, 0), n - 1)
                vals = plsc.load_gather(buf, [idx_safe])
                vals = jnp.where(valid, vals, inf_v)
                idxm = jnp.where(valid, idx, imax_v)
                sv, si = plsc.sort_key_val(vals, idxm)
                rv, ri = lax.rev(sv, (0,)), lax.rev(si, (0,))
                lt = (rv < c1v) | ((rv == c1v) & (ri < c1i))
                lov = jnp.where(lt, rv, c1v)
                loi = jnp.where(lt, ri, c1i)
                lov, loi = plsc.sort_key_val(lov, loi)
                rv2, ri2 = lax.rev(lov, (0,)), lax.rev(loi, (0,))
                lt2 = (rv2 < c0v) | ((rv2 == c0v) & (ri2 < c0i))
                n0v = jnp.where(lt2, rv2, c0v)
                n0i = jnp.where(lt2, ri2, c0i)
                n1v = jnp.where(lt2, c0v, rv2)
                n1i = jnp.where(lt2, c0i, ri2)
                n0v, n0i = plsc.sort_key_val(n0v, n0i)
                n1v, n1i = plsc.sort_key_val(n1v, n1i)
                return n0v, n0i, n1v, n1i

            init = (inf_v, imax_v, inf_v, imax_v)
            _, c0i, _, c1i = lax.fori_loop(0, nb, sel_body, init)
            obuf[pl.ds(0, 16)] = c0i
            obuf[pl.ds(16, 16)] = c1i
            oall[pl.ds(r * K, 16)] = plsc.load_gather(obuf, [iota16 * 2])

        def outer(k2, carry):
            r0 = 2 * k2
            row = base + r0
            pltpu.async_copy(dist_hbm.at[pl.ds((row + 1) * n, n)], buf_b, sem_b)
            pltpu.make_async_copy(dist_hbm.at[pl.ds(row * n, n)], buf_a, sem_a).wait()
            process(buf_a, r0)

            @pl.when(r0 + 2 < rpw)
            def _():
                pltpu.async_copy(dist_hbm.at[pl.ds((row + 2) * n, n)], buf_a, sem_a)

            pltpu.make_async_copy(dist_hbm.at[pl.ds((row + 1) * n, n)], buf_b, sem_b).wait()
            process(buf_b, r0 + 1)
            return carry

        lax.fori_loop(0, rpw // 2, outer, jnp.int32(0))
        pltpu.sync_copy(oall, out_hbm.at[pl.ds(base * K, rpw * K)])

    return select


def kernel(x):
    B, C, N, _ = x.shape
    xt = jnp.squeeze(jnp.swapaxes(x, 2, 1), -1)
    dist, gmins = _pairwise_dist(xt)
    sel = _make_select(B * N, N)
    nn_idx = sel(dist.reshape(B * N * N), gmins.reshape(B * N * 32))
    nn_idx = nn_idx.reshape(B, N, K)
    center = jnp.broadcast_to(
        jnp.arange(N, dtype=nn_idx.dtype)[None, :, None], (B, N, K)
    )
    return jnp.stack((nn_idx, center), axis=0)


# speedup vs baseline: 9.7926x; 1.2834x over previous
import functools

import jax
import jax.numpy as jnp
from jax import lax
from jax.experimental import pallas as pl
from jax.experimental.pallas import tpu as pltpu
from jax.experimental.pallas import tpu_sc as plsc

K = 16
DILATION = 2
KSEL = K * DILATION

NC = 2
NS = 16
NW = NC * NS
INT_MAX = 2147483647


def _dist_body(xl_ref, xr_ref, d_ref, g_ref):
    a = xl_ref[0]
    b = xr_ref[0]
    mm = jax.lax.dot_general(
        a, b, (((1,), (1,)), ((), ())), preferred_element_type=jnp.float32
    )
    asq = jnp.sum(a * a, axis=1, keepdims=True)
    bsq = jnp.sum(b * b, axis=1, keepdims=True)
    d = (asq + (-2.0 * mm)) + bsq.T
    d_ref[0] = d
    bi, n = d.shape
    g_ref[0] = jnp.min(d.reshape(bi, 32, n // 32), axis=2)


@functools.partial(jax.jit, static_argnames=("bi",))
def _pairwise_dist(xt, bi=256):
    B, N, C = xt.shape
    grid = (B, N // bi)
    return pl.pallas_call(
        _dist_body,
        grid=grid,
        in_specs=[
            pl.BlockSpec((1, bi, C), lambda b, i: (b, i, 0)),
            pl.BlockSpec((1, N, C), lambda b, i: (b, 0, 0)),
        ],
        out_specs=[
            pl.BlockSpec((1, bi, N), lambda b, i: (b, i, 0)),
            pl.BlockSpec((1, bi, 32), lambda b, i: (b, i, 0)),
        ],
        out_shape=[
            jax.ShapeDtypeStruct((B, N, N), jnp.float32),
            jax.ShapeDtypeStruct((B, N, 32), jnp.float32),
        ],
        compiler_params=pltpu.CompilerParams(
            dimension_semantics=("parallel", "arbitrary"),
        ),
    )(xt, xt)


def _make_select(num_rows, n):
    rpw = num_rows // NW
    nvreg = n // 16
    cap = 1024 + 32
    mesh = plsc.VectorSubcoreMesh(
        core_axis_name="c", subcore_axis_name="s", num_cores=NC, num_subcores=NS
    )

    @functools.partial(
        pl.kernel,
        out_type=jax.ShapeDtypeStruct((num_rows * K,), jnp.int32),
        mesh=mesh,
        compiler_params=pltpu.CompilerParams(needs_layout_passes=False),
        scratch_types=[
            pltpu.VMEM((n,), jnp.float32),
            pltpu.VMEM((n,), jnp.float32),
            pltpu.VMEM((rpw * 32,), jnp.float32),
            pltpu.VMEM((cap,), jnp.int32),
            pltpu.VMEM((32,), jnp.int32),
            pltpu.VMEM((rpw * K,), jnp.int32),
            pltpu.SemaphoreType.DMA,
            pltpu.SemaphoreType.DMA,
        ],
    )
    def select(dist_hbm, gmins_hbm, out_hbm, buf_a, buf_b, gall, cand, obuf,
               oall, sem_a, sem_b):
        wid = lax.axis_index("s") * NC + lax.axis_index("c")
        base = wid * rpw
        iota16 = lax.broadcasted_iota(jnp.int32, (16,), 0)
        inf_v = jnp.full((16,), jnp.inf, jnp.float32)
        imax_v = jnp.full((16,), INT_MAX, jnp.int32)

        pltpu.sync_copy(gmins_hbm.at[pl.ds(base * 32, rpw * 32)], gall)
        pltpu.async_copy(dist_hbm.at[base], buf_a, sem_a)

        def process(buf, r):
            gm = jnp.maximum(
                gall[pl.ds(r * 32, 16)], gall[pl.ds(r * 32 + 16, 16)]
            )
            theta = jnp.max(gm, axis=0)
            theta_v = jnp.full((16,), theta, jnp.float32)

            def scan_body(g, cnt):
                x = buf[pl.ds(g * 16, 16)]
                m = x <= theta_v
                plsc.store_compressed(
                    cand.at[pl.ds(cnt, 16)], iota16 + g * 16, mask=m
                )
                return cnt + plsc.all_reduce_population_count(m)[0]

            cnt = lax.fori_loop(0, nvreg, scan_body, jnp.int32(0), unroll=8)

            nb = (cnt + 15) >> 4
            cnt_v = jnp.full((16,), cnt, jnp.int32)

            def sel_body(t, carry):
                c0v, c0i, c1v, c1i = carry
                off = t * 16
                idx = cand[pl.ds(off, 16)]
                valid = (iota16 + off) < cnt_v
                idx_safe = jnp.minimum(jnp.maximum(idx, 0), n - 1)
                vals = plsc.load_gather(buf, [idx_safe])
                vals = jnp.where(valid, vals, inf_v)
                idxm = jnp.where(valid, idx, imax_v)
                sv, si = plsc.sort_key_val(vals, idxm)
                rv, ri = lax.rev(sv, (0,)), lax.rev(si, (0,))
                lt = (rv < c1v) | ((rv == c1v) & (ri < c1i))
                lov = jnp.where(lt, rv, c1v)
                loi = jnp.where(lt, ri, c1i)
                lov, loi = plsc.sort_key_val(lov, loi)
                rv2, ri2 = lax.rev(lov, (0,)), lax.rev(loi, (0,))
                lt2 = (rv2 < c0v) | ((rv2 == c0v) & (ri2 < c0i))
                n0v = jnp.where(lt2, rv2, c0v)
                n0i = jnp.where(lt2, ri2, c0i)
                n1v = jnp.where(lt2, c0v, rv2)
                n1i = jnp.where(lt2, c0i, ri2)
                n0v, n0i = plsc.sort_key_val(n0v, n0i)
                n1v, n1i = plsc.sort_key_val(n1v, n1i)
                return n0v, n0i, n1v, n1i

            init = (inf_v, imax_v, inf_v, imax_v)
            _, c0i, _, c1i = lax.fori_loop(0, nb, sel_body, init)
            obuf[pl.ds(0, 16)] = c0i
            obuf[pl.ds(16, 16)] = c1i
            oall[pl.ds(r * K, 16)] = plsc.load_gather(obuf, [iota16 * 2])

        def outer(k2, carry):
            r0 = 2 * k2
            row = base + r0
            pltpu.async_copy(dist_hbm.at[row + 1], buf_b, sem_b)
            pltpu.make_async_copy(dist_hbm.at[row], buf_a, sem_a).wait()
            process(buf_a, r0)

            @pl.when(r0 + 2 < rpw)
            def _():
                pltpu.async_copy(dist_hbm.at[row + 2], buf_a, sem_a)

            pltpu.make_async_copy(dist_hbm.at[row + 1], buf_b, sem_b).wait()
            process(buf_b, r0 + 1)
            return carry

        lax.fori_loop(0, rpw // 2, outer, jnp.int32(0))
        pltpu.sync_copy(oall, out_hbm.at[pl.ds(base * K, rpw * K)])

    return select


def kernel(x):
    B, C, N, _ = x.shape
    xt = jnp.squeeze(jnp.swapaxes(x, 2, 1), -1)
    dist, gmins = _pairwise_dist(xt)
    sel = _make_select(B * N, N)
    nn_idx = sel(dist.reshape(B * N, N), gmins.reshape(B * N * 32))
    nn_idx = nn_idx.reshape(B, N, K)
    center = jnp.broadcast_to(
        jnp.arange(N, dtype=nn_idx.dtype)[None, :, None], (B, N, K)
    )
    return jnp.stack((nn_idx, center), axis=0)
